# Initial kernel scaffold; baseline (speedup 1.0000x reference)
import jax
import jax.numpy as jnp
from jax.experimental import pallas as pl


def kernel(input_features, expert_centroids, ln_g, ln_b, w1, b1, w2, b2):
    raise NotImplementedError("write your pallas kernel here")


# speedup vs baseline: 1.5723x; 1.5723x over previous
import functools

import jax
import jax.numpy as jnp
from jax import lax
from jax.experimental import pallas as pl
from jax.experimental.pallas import tpu as pltpu
from jax.experimental.pallas import tpu_sc as plsc

_E = 8
_D = 1024
_F = 4096
_S = 2048
_TM = 128
_FT = 512
_NF = _F // _FT
_NT = _S // _TM + _E
_NP = _NT * _TM
_TR = _S // 128
_W = 32


def _scores_body(x_ref, c_ref, o_ref):
    o_ref[...] = lax.dot_general(
        x_ref[...], c_ref[...], (((1,), (1,)), ((), ())),
        preferred_element_type=jnp.float32)


def _route_body(st_ref, slot_ref, te_ref, nt_ref):
    s = [st_ref[e] for e in range(_E)]
    best = s[0]
    for e in range(1, _E):
        best = jnp.maximum(best, s[e])
    bi = jnp.full((_TR, 128), _E - 1, jnp.int32)
    for e in range(_E - 2, -1, -1):
        bi = jnp.where(s[e] == best, e, bi)

    r = lax.broadcasted_iota(jnp.int32, (128, 128), 0)
    c = lax.broadcasted_iota(jnp.int32, (128, 128), 1)
    upper = (r <= c).astype(jnp.float32)
    r16 = lax.broadcasted_iota(jnp.int32, (_TR, _TR), 0)
    c16 = lax.broadcasted_iota(jnp.int32, (_TR, _TR), 1)
    lower = (c16 < r16).astype(jnp.float32)

    iota_nt = lax.broadcasted_iota(jnp.int32, (1, _NT), 1)
    slot = jnp.zeros((_TR, 128), jnp.float32)
    te = jnp.zeros((1, _NT), jnp.int32)
    pad_off = jnp.zeros((1, 1), jnp.int32)
    cum_tiles = jnp.zeros((1, 1), jnp.int32)
    last_e = jnp.zeros((1, 1), jnp.int32)
    for e in range(_E):
        m = (bi == e).astype(jnp.float32)
        incl = lax.dot_general(m, upper, (((1,), (0,)), ((), ())),
                               preferred_element_type=jnp.float32)
        rowsum = jnp.sum(m, axis=1, keepdims=True)
        rowoff = lax.dot_general(lower, rowsum, (((1,), (0,)), ((), ())),
                                 preferred_element_type=jnp.float32)
        rank = incl - 1.0 + rowoff
        slot = slot + m * (pad_off.astype(jnp.float32) + rank)
        cnt = jnp.sum(m, axis=(0, 1), keepdims=True).astype(jnp.int32)
        ntile = (cnt + _TM - 1) // _TM
        last_e = jnp.where(cnt > 0, e, last_e)
        cum_tiles = cum_tiles + ntile
        te = te + (iota_nt >= cum_tiles).astype(jnp.int32)
        pad_off = pad_off + ntile * _TM
    te = jnp.where(iota_nt < cum_tiles, te, last_e)
    slot_ref[...] = slot.astype(jnp.int32)
    te_ref[...] = te
    nt_ref[...] = cum_tiles


def _ffn_body(te_ref, nt_ref, x_ref, w1_ref, b1_ref, w2_ref, b2_ref,
              g_ref, bb_ref, c_ref, o_ref, acc_ref):
    f = pl.program_id(0)
    u = pl.program_id(1)
    total = nt_ref[0]

    @pl.when(u < total)
    def _():
        x = x_ref[...]
        mu = jnp.mean(x, axis=1, keepdims=True)
        xc = x - mu
        var = jnp.mean(xc * xc, axis=1, keepdims=True)
        ln = xc * lax.rsqrt(var + 1e-5) * g_ref[0] + bb_ref[0]
        h1 = lax.dot_general(ln, w1_ref[0], (((1,), (1,)), ((), ())),
                             preferred_element_type=jnp.float32)
        h1 = jnp.maximum(h1 + b1_ref[0, 0], 0.0)
        h2 = lax.dot_general(h1, w2_ref[0], (((1,), (1,)), ((), ())),
                             preferred_element_type=jnp.float32)
        base = pl.multiple_of(u * _TM, _TM)

        @pl.when(f == 0)
        def _():
            acc_ref[pl.ds(base, _TM), :] = h2

        @pl.when(f > 0)
        def _():
            acc_ref[pl.ds(base, _TM), :] += h2

        @pl.when(f == _NF - 1)
        def _():
            z = jnp.sum(x * c_ref[0], axis=1, keepdims=True)
            alpha = 1.0 / (1.0 + jnp.exp(-z))
            o_ref[...] = x + alpha * (acc_ref[pl.ds(base, _TM), :]
                                      + b2_ref[0])


def _ueff(u, nt):
    return jnp.minimum(u, nt[0] - 1)


def _sc_mesh():
    return plsc.VectorSubcoreMesh(core_axis_name="core",
                                  subcore_axis_name="subcore")


_NW = 32
_BW = _S // _NW


def _dispatch(x, slot):
    @functools.partial(
        pl.kernel,
        out_type=jax.ShapeDtypeStruct((_NP, _D), jnp.float32),
        mesh=_sc_mesh(),
        scratch_types=[pltpu.VMEM((_BW,), jnp.int32),
                       pltpu.VMEM((_BW, _D), jnp.float32)])
    def kern(x_hbm, s_hbm, o_hbm, idx_v, rows_v):
        wid = lax.axis_index("subcore") * 2 + lax.axis_index("core")
        base = wid * _BW
        pltpu.sync_copy(s_hbm.at[pl.ds(base, _BW)], idx_v)
        pltpu.sync_copy(x_hbm.at[pl.ds(base, _BW)], rows_v)
        pltpu.sync_copy(rows_v, o_hbm.at[idx_v])

    return kern(x, slot)


def _combine(out_pad, slot):
    @functools.partial(
        pl.kernel,
        out_type=jax.ShapeDtypeStruct((_S, _D), jnp.float32),
        mesh=_sc_mesh(),
        scratch_types=[pltpu.VMEM((_BW,), jnp.int32),
                       pltpu.VMEM((_BW, _D), jnp.float32)])
    def kern(p_hbm, s_hbm, y_hbm, idx_v, rows_v):
        wid = lax.axis_index("subcore") * 2 + lax.axis_index("core")
        base = wid * _BW
        pltpu.sync_copy(s_hbm.at[pl.ds(base, _BW)], idx_v)
        pltpu.sync_copy(p_hbm.at[idx_v], rows_v)
        pltpu.sync_copy(rows_v, y_hbm.at[pl.ds(base, _BW)])

    return kern(out_pad, slot)


def _routing(x, expert_centroids):
    scores = pl.pallas_call(
        _scores_body,
        grid=(8,),
        in_specs=[pl.BlockSpec((_S // 8, _D), lambda i: (i, 0)),
                  pl.BlockSpec((_E, _D), lambda i: (0, 0))],
        out_specs=pl.BlockSpec((_S // 8, _E), lambda i: (i, 0)),
        out_shape=jax.ShapeDtypeStruct((_S, _E), jnp.float32),
    )(x, expert_centroids)
    st = scores.T.reshape(_E, _TR, 128)
    slot2d, te2d, nt2d = pl.pallas_call(
        _route_body,
        out_shape=(
            jax.ShapeDtypeStruct((_TR, 128), jnp.int32),
            jax.ShapeDtypeStruct((1, _NT), jnp.int32),
            jax.ShapeDtypeStruct((1, 1), jnp.int32),
        ),
    )(st)
    return slot2d.reshape(_S), te2d.reshape(_NT), nt2d.reshape(1)


def _ffn(te, nt, x_pad, w1, b1, w2, b2, ln_g, ln_b, cent):
    grid_spec = pltpu.PrefetchScalarGridSpec(
        num_scalar_prefetch=2,
        grid=(_NF, _NT),
        in_specs=[
            pl.BlockSpec((_TM, _D), lambda f, u, te, nt: (_ueff(u, nt), 0)),
            pl.BlockSpec((1, _FT, _D),
                         lambda f, u, te, nt: (te[_ueff(u, nt)], f, 0)),
            pl.BlockSpec((1, 1, 1, _FT),
                         lambda f, u, te, nt: (te[_ueff(u, nt)], f, 0, 0)),
            pl.BlockSpec((1, _D, _FT),
                         lambda f, u, te, nt: (te[_ueff(u, nt)], 0, f)),
            pl.BlockSpec((1, 1, _D),
                         lambda f, u, te, nt: (te[_ueff(u, nt)], 0, 0)),
            pl.BlockSpec((1, 1, _D),
                         lambda f, u, te, nt: (te[_ueff(u, nt)], 0, 0)),
            pl.BlockSpec((1, 1, _D),
                         lambda f, u, te, nt: (te[_ueff(u, nt)], 0, 0)),
            pl.BlockSpec((1, 1, _D),
                         lambda f, u, te, nt: (te[_ueff(u, nt)], 0, 0)),
        ],
        out_specs=pl.BlockSpec(
            (_TM, _D),
            lambda f, u, te, nt: (jnp.where(f == _NF - 1, _ueff(u, nt), 0), 0)),
        scratch_shapes=[pltpu.VMEM((_NP, _D), jnp.float32)],
    )
    return pl.pallas_call(
        _ffn_body,
        grid_spec=grid_spec,
        out_shape=jax.ShapeDtypeStruct((_NP, _D), jnp.float32),
        compiler_params=pltpu.CompilerParams(
            dimension_semantics=("arbitrary", "arbitrary"),
            vmem_limit_bytes=60 * 1024 * 1024),
    )(te, nt, x_pad, w1, b1.reshape(_E, _NF, 1, _FT), w2,
      b2.reshape(_E, 1, _D), ln_g.reshape(_E, 1, _D),
      ln_b.reshape(_E, 1, _D), cent.reshape(_E, 1, _D))


def kernel(input_features, expert_centroids, ln_g, ln_b, w1, b1, w2, b2):
    x = input_features.reshape(_S, _D)
    slot_row, te, nt = _routing(x, expert_centroids)
    x_pad = _dispatch(x, slot_row)
    out_pad = _ffn(te, nt, x_pad, w1, b1, w2, b2, ln_g, ln_b,
                   expert_centroids)
    y = _combine(out_pad, slot_row)
    return y.reshape(input_features.shape)


# speedup vs baseline: 1.9638x; 1.2490x over previous
import functools

import jax
import jax.numpy as jnp
from jax import lax
from jax.experimental import pallas as pl
from jax.experimental.pallas import tpu as pltpu
from jax.experimental.pallas import tpu_sc as plsc

_E = 8
_D = 1024
_F = 4096
_S = 2048
_TM = 128
_FT = 1024
_NF = _F // _FT
_NT = _S // _TM + _E
_NP = _NT * _TM
_TR = _S // 128
_W = 32


def _scores_body(x_ref, c_ref, o_ref):
    o_ref[...] = lax.dot_general(
        x_ref[...], c_ref[...], (((1,), (1,)), ((), ())),
        preferred_element_type=jnp.float32)


def _route_body(st_ref, slot_ref, te_ref, nt_ref):
    s = [st_ref[e] for e in range(_E)]
    best = s[0]
    for e in range(1, _E):
        best = jnp.maximum(best, s[e])
    bi = jnp.full((_TR, 128), _E - 1, jnp.int32)
    for e in range(_E - 2, -1, -1):
        bi = jnp.where(s[e] == best, e, bi)

    r = lax.broadcasted_iota(jnp.int32, (128, 128), 0)
    c = lax.broadcasted_iota(jnp.int32, (128, 128), 1)
    upper = (r <= c).astype(jnp.float32)
    r16 = lax.broadcasted_iota(jnp.int32, (_TR, _TR), 0)
    c16 = lax.broadcasted_iota(jnp.int32, (_TR, _TR), 1)
    lower = (c16 < r16).astype(jnp.float32)

    iota_nt = lax.broadcasted_iota(jnp.int32, (1, _NT), 1)
    slot = jnp.zeros((_TR, 128), jnp.float32)
    te = jnp.zeros((1, _NT), jnp.int32)
    pad_off = jnp.zeros((1, 1), jnp.int32)
    cum_tiles = jnp.zeros((1, 1), jnp.int32)
    last_e = jnp.zeros((1, 1), jnp.int32)
    for e in range(_E):
        m = (bi == e).astype(jnp.float32)
        incl = lax.dot_general(m, upper, (((1,), (0,)), ((), ())),
                               preferred_element_type=jnp.float32)
        rowsum = jnp.sum(m, axis=1, keepdims=True)
        rowoff = lax.dot_general(lower, rowsum, (((1,), (0,)), ((), ())),
                                 preferred_element_type=jnp.float32)
        rank = incl - 1.0 + rowoff
        slot = slot + m * (pad_off.astype(jnp.float32) + rank)
        cnt = jnp.sum(m, axis=(0, 1), keepdims=True).astype(jnp.int32)
        ntile = (cnt + _TM - 1) // _TM
        last_e = jnp.where(cnt > 0, e, last_e)
        cum_tiles = cum_tiles + ntile
        te = te + (iota_nt >= cum_tiles).astype(jnp.int32)
        pad_off = pad_off + ntile * _TM
    te = jnp.where(iota_nt < cum_tiles, te, last_e)
    slot_ref[...] = slot.astype(jnp.int32)
    te_ref[...] = te
    nt_ref[...] = cum_tiles


def _ffn_body(te_ref, nt_ref, x_ref, w1_ref, b1_ref, w2_ref, b2_ref,
              g_ref, bb_ref, c_ref, o_ref,
              acc_ref, lnb_ref, alpha_ref, w1b_ref, w2b_ref):
    f = pl.program_id(0)
    u = pl.program_id(1)
    total = nt_ref[0]

    @pl.when(u < total)
    def _():
        base = pl.multiple_of(u * _TM, _TM)
        sl = (pl.ds(base, _TM), slice(None))

        new_w = jnp.logical_or(u == 0,
                               te_ref[jnp.maximum(u, 1) - 1] != te_ref[u])

        @pl.when(new_w)
        def _():
            w1b_ref[...] = w1_ref[0].astype(jnp.bfloat16)
            w2b_ref[...] = w2_ref[0].astype(jnp.bfloat16)

        @pl.when(f == 0)
        def _():
            x = x_ref[...]
            mu = jnp.mean(x, axis=1, keepdims=True)
            xc = x - mu
            var = jnp.mean(xc * xc, axis=1, keepdims=True)
            ln = xc * lax.rsqrt(var + 1e-5) * g_ref[0] + bb_ref[0]
            lnb_ref[sl] = ln.astype(jnp.bfloat16)
            z = jnp.sum(x * c_ref[0], axis=1, keepdims=True)
            alpha = 1.0 / (1.0 + jnp.exp(-z))
            alpha_ref[sl] = alpha
            acc_ref[sl] = x + alpha * b2_ref[0]

        h1 = lax.dot_general(lnb_ref[sl], w1b_ref[...],
                             (((1,), (1,)), ((), ())),
                             preferred_element_type=jnp.float32)
        h1 = jnp.maximum(h1 + b1_ref[0, 0], 0.0).astype(jnp.bfloat16)
        h2 = lax.dot_general(h1, w2b_ref[...], (((1,), (1,)), ((), ())),
                             preferred_element_type=jnp.float32)
        upd = acc_ref[sl] + alpha_ref[sl] * h2

        @pl.when(f < _NF - 1)
        def _():
            acc_ref[sl] = upd

        @pl.when(f == _NF - 1)
        def _():
            o_ref[...] = upd


def _ueff(u, nt):
    return jnp.minimum(u, nt[0] - 1)


def _sc_mesh():
    return plsc.VectorSubcoreMesh(core_axis_name="core",
                                  subcore_axis_name="subcore")


_NW = 32
_BW = _S // _NW


def _dispatch(x, slot):
    @functools.partial(
        pl.kernel,
        out_type=jax.ShapeDtypeStruct((_NP, _D), jnp.float32),
        mesh=_sc_mesh(),
        scratch_types=[pltpu.VMEM((_BW,), jnp.int32),
                       pltpu.VMEM((_BW, _D), jnp.float32)])
    def kern(x_hbm, s_hbm, o_hbm, idx_v, rows_v):
        wid = lax.axis_index("subcore") * 2 + lax.axis_index("core")
        base = wid * _BW
        pltpu.sync_copy(s_hbm.at[pl.ds(base, _BW)], idx_v)
        pltpu.sync_copy(x_hbm.at[pl.ds(base, _BW)], rows_v)
        pltpu.sync_copy(rows_v, o_hbm.at[idx_v])

    return kern(x, slot)


def _combine(out_pad, slot):
    @functools.partial(
        pl.kernel,
        out_type=jax.ShapeDtypeStruct((_S, _D), jnp.float32),
        mesh=_sc_mesh(),
        scratch_types=[pltpu.VMEM((_BW,), jnp.int32),
                       pltpu.VMEM((_BW, _D), jnp.float32)])
    def kern(p_hbm, s_hbm, y_hbm, idx_v, rows_v):
        wid = lax.axis_index("subcore") * 2 + lax.axis_index("core")
        base = wid * _BW
        pltpu.sync_copy(s_hbm.at[pl.ds(base, _BW)], idx_v)
        pltpu.sync_copy(p_hbm.at[idx_v], rows_v)
        pltpu.sync_copy(rows_v, y_hbm.at[pl.ds(base, _BW)])

    return kern(out_pad, slot)


def _routing(x, expert_centroids):
    scores = pl.pallas_call(
        _scores_body,
        grid=(8,),
        in_specs=[pl.BlockSpec((_S // 8, _D), lambda i: (i, 0)),
                  pl.BlockSpec((_E, _D), lambda i: (0, 0))],
        out_specs=pl.BlockSpec((_S // 8, _E), lambda i: (i, 0)),
        out_shape=jax.ShapeDtypeStruct((_S, _E), jnp.float32),
    )(x, expert_centroids)
    st = scores.T.reshape(_E, _TR, 128)
    slot2d, te2d, nt2d = pl.pallas_call(
        _route_body,
        out_shape=(
            jax.ShapeDtypeStruct((_TR, 128), jnp.int32),
            jax.ShapeDtypeStruct((1, _NT), jnp.int32),
            jax.ShapeDtypeStruct((1, 1), jnp.int32),
        ),
    )(st)
    return slot2d.reshape(_S), te2d.reshape(_NT), nt2d.reshape(1)


def _ffn(te, nt, x_pad, w1, b1, w2, b2, ln_g, ln_b, cent):
    grid_spec = pltpu.PrefetchScalarGridSpec(
        num_scalar_prefetch=2,
        grid=(_NF, _NT),
        in_specs=[
            pl.BlockSpec((_TM, _D),
                         lambda f, u, te, nt: (
                             jnp.where(f == 0, _ueff(u, nt), 0), 0)),
            pl.BlockSpec((1, _FT, _D),
                         lambda f, u, te, nt: (te[_ueff(u, nt)], f, 0)),
            pl.BlockSpec((1, 1, 1, _FT),
                         lambda f, u, te, nt: (te[_ueff(u, nt)], f, 0, 0)),
            pl.BlockSpec((1, _D, _FT),
                         lambda f, u, te, nt: (te[_ueff(u, nt)], 0, f)),
            pl.BlockSpec((1, 1, _D),
                         lambda f, u, te, nt: (te[_ueff(u, nt)], 0, 0)),
            pl.BlockSpec((1, 1, _D),
                         lambda f, u, te, nt: (te[_ueff(u, nt)], 0, 0)),
            pl.BlockSpec((1, 1, _D),
                         lambda f, u, te, nt: (te[_ueff(u, nt)], 0, 0)),
            pl.BlockSpec((1, 1, _D),
                         lambda f, u, te, nt: (te[_ueff(u, nt)], 0, 0)),
        ],
        out_specs=pl.BlockSpec(
            (_TM, _D),
            lambda f, u, te, nt: (jnp.where(f == _NF - 1, _ueff(u, nt), 0), 0)),
        scratch_shapes=[pltpu.VMEM((_NP, _D), jnp.float32),
                        pltpu.VMEM((_NP, _D), jnp.bfloat16),
                        pltpu.VMEM((_NP, 1), jnp.float32),
                        pltpu.VMEM((_FT, _D), jnp.bfloat16),
                        pltpu.VMEM((_D, _FT), jnp.bfloat16)],
    )
    return pl.pallas_call(
        _ffn_body,
        grid_spec=grid_spec,
        out_shape=jax.ShapeDtypeStruct((_NP, _D), jnp.float32),
        compiler_params=pltpu.CompilerParams(
            dimension_semantics=("arbitrary", "arbitrary"),
            vmem_limit_bytes=60 * 1024 * 1024),
    )(te, nt, x_pad, w1, b1.reshape(_E, _NF, 1, _FT), w2,
      b2.reshape(_E, 1, _D), ln_g.reshape(_E, 1, _D),
      ln_b.reshape(_E, 1, _D), cent.reshape(_E, 1, _D))


def kernel(input_features, expert_centroids, ln_g, ln_b, w1, b1, w2, b2):
    x = input_features.reshape(_S, _D)
    slot_row, te, nt = _routing(x, expert_centroids)
    x_pad = _dispatch(x, slot_row)
    out_pad = _ffn(te, nt, x_pad, w1, b1, w2, b2, ln_g, ln_b,
                   expert_centroids)
    y = _combine(out_pad, slot_row)
    return y.reshape(input_features.shape)


# speedup vs baseline: 2.2956x; 1.1690x over previous
import functools

import jax
import jax.numpy as jnp
from jax import lax
from jax.experimental import pallas as pl
from jax.experimental.pallas import tpu as pltpu
from jax.experimental.pallas import tpu_sc as plsc

_E = 8
_D = 1024
_F = 4096
_S = 2048
_TM = 128
_FT = 1024
_NF = _F // _FT
_NT = _S // _TM + _E
_NP = _NT * _TM
_TR = _S // 128


def _scores_body(x_ref, c_ref, o_ref):
    o_ref[...] = lax.dot_general(
        x_ref[...], c_ref[...], (((1,), (1,)), ((), ())),
        preferred_element_type=jnp.float32)


def _route_body(st_ref, slot_ref, te_ref, nt_ref, cb_ref, al_ref):
    s = [st_ref[e] for e in range(_E)]
    best = s[0]
    for e in range(1, _E):
        best = jnp.maximum(best, s[e])
    bi = jnp.full((_TR, 128), _E - 1, jnp.int32)
    for e in range(_E - 2, -1, -1):
        bi = jnp.where(s[e] == best, e, bi)
    al_ref[...] = 1.0 / (1.0 + jnp.exp(-best))

    r = lax.broadcasted_iota(jnp.int32, (128, 128), 0)
    c = lax.broadcasted_iota(jnp.int32, (128, 128), 1)
    upper = (r <= c).astype(jnp.float32)
    r16 = lax.broadcasted_iota(jnp.int32, (_TR, _TR), 0)
    c16 = lax.broadcasted_iota(jnp.int32, (_TR, _TR), 1)
    lower = (c16 < r16).astype(jnp.float32)

    iota_nt = lax.broadcasted_iota(jnp.int32, (1, _NT), 1)
    iota_e1 = lax.broadcasted_iota(jnp.int32, (1, _E + 1), 1)
    slot = jnp.zeros((_TR, 128), jnp.float32)
    te = jnp.zeros((1, _NT), jnp.int32)
    cb = jnp.zeros((1, _E + 1), jnp.int32)
    pad_off = jnp.zeros((1, 1), jnp.int32)
    cum_tiles = jnp.zeros((1, 1), jnp.int32)
    last_e = jnp.zeros((1, 1), jnp.int32)
    for e in range(_E):
        m = (bi == e).astype(jnp.float32)
        incl = lax.dot_general(m, upper, (((1,), (0,)), ((), ())),
                               preferred_element_type=jnp.float32)
        rowsum = jnp.sum(m, axis=1, keepdims=True)
        rowoff = lax.dot_general(lower, rowsum, (((1,), (0,)), ((), ())),
                                 preferred_element_type=jnp.float32)
        rank = incl - 1.0 + rowoff
        slot = slot + m * (pad_off.astype(jnp.float32) + rank)
        cnt = jnp.sum(m, axis=(0, 1), keepdims=True).astype(jnp.int32)
        ntile = (cnt + _TM - 1) // _TM
        last_e = jnp.where(cnt > 0, e, last_e)
        cum_tiles = cum_tiles + ntile
        te = te + (iota_nt >= cum_tiles).astype(jnp.int32)
        cb = cb + (iota_e1 == e + 1) * cum_tiles
        pad_off = pad_off + ntile * _TM
    te = jnp.where(iota_nt < cum_tiles, te, last_e)
    slot_ref[...] = slot.astype(jnp.int32)
    te_ref[...] = te
    nt_ref[...] = cum_tiles
    cb_ref[...] = cb


def _ln_body(te_ref, nt_ref, x_ref, g_ref, bb_ref, o_ref):
    u = pl.program_id(0)

    @pl.when(u < nt_ref[0])
    def _():
        x = x_ref[...]
        mu = jnp.mean(x, axis=1, keepdims=True)
        xc = x - mu
        var = jnp.mean(xc * xc, axis=1, keepdims=True)
        ln = xc * lax.rsqrt(var + 1e-5) * g_ref[0] + bb_ref[0]
        o_ref[...] = ln.astype(jnp.bfloat16)


def _ffn_body(cb_ref, w1_ref, b1_ref, w2_ref, b2_ref, lnb_ref, o_ref,
              w1b_ref, w2b_ref):
    f = pl.program_id(0)
    e = pl.program_id(1)
    w1b_ref[...] = w1_ref[0].astype(jnp.bfloat16)
    w2b_ref[...] = w2_ref[0].astype(jnp.bfloat16)

    def tile_body(t, carry):
        base = pl.multiple_of(t * _TM, _TM)
        sl = (pl.ds(base, _TM), slice(None))
        h1 = lax.dot_general(lnb_ref[sl], w1b_ref[...],
                             (((1,), (1,)), ((), ())),
                             preferred_element_type=jnp.float32)
        h1 = jnp.maximum(h1 + b1_ref[0, 0], 0.0).astype(jnp.bfloat16)
        h2 = lax.dot_general(h1, w2b_ref[...], (((1,), (1,)), ((), ())),
                             preferred_element_type=jnp.float32)

        @pl.when(f == 0)
        def _():
            o_ref[sl] = h2 + b2_ref[0]

        @pl.when(f > 0)
        def _():
            o_ref[sl] += h2

        return carry

    lax.fori_loop(cb_ref[e], cb_ref[e + 1], tile_body, 0)


def _resid_body(x_ref, g_ref, a_ref, o_ref):
    o_ref[...] = x_ref[...] + a_ref[...] * g_ref[...]


def _sc_mesh():
    return plsc.VectorSubcoreMesh(core_axis_name="core",
                                  subcore_axis_name="subcore")


_NW = 32
_BW = _S // _NW


def _dispatch(x, slot):
    @functools.partial(
        pl.kernel,
        out_type=jax.ShapeDtypeStruct((_NP, _D), jnp.float32),
        mesh=_sc_mesh(),
        scratch_types=[pltpu.VMEM((_BW,), jnp.int32),
                       pltpu.VMEM((_BW, _D), jnp.float32)])
    def kern(x_hbm, s_hbm, o_hbm, idx_v, rows_v):
        wid = lax.axis_index("subcore") * 2 + lax.axis_index("core")
        base = wid * _BW
        pltpu.sync_copy(s_hbm.at[pl.ds(base, _BW)], idx_v)
        pltpu.sync_copy(x_hbm.at[pl.ds(base, _BW)], rows_v)
        pltpu.sync_copy(rows_v, o_hbm.at[idx_v])

    return kern(x, slot)


def _combine(ffn_pad, slot):
    @functools.partial(
        pl.kernel,
        out_type=jax.ShapeDtypeStruct((_S, _D), jnp.float32),
        mesh=_sc_mesh(),
        scratch_types=[pltpu.VMEM((_BW,), jnp.int32),
                       pltpu.VMEM((_BW, _D), jnp.float32)])
    def kern(p_hbm, s_hbm, y_hbm, idx_v, rows_v):
        wid = lax.axis_index("subcore") * 2 + lax.axis_index("core")
        base = wid * _BW
        pltpu.sync_copy(s_hbm.at[pl.ds(base, _BW)], idx_v)
        pltpu.sync_copy(p_hbm.at[idx_v], rows_v)
        pltpu.sync_copy(rows_v, y_hbm.at[pl.ds(base, _BW)])

    return kern(ffn_pad, slot)


def _routing(x, expert_centroids):
    scores = pl.pallas_call(
        _scores_body,
        grid=(8,),
        in_specs=[pl.BlockSpec((_S // 8, _D), lambda i: (i, 0)),
                  pl.BlockSpec((_E, _D), lambda i: (0, 0))],
        out_specs=pl.BlockSpec((_S // 8, _E), lambda i: (i, 0)),
        out_shape=jax.ShapeDtypeStruct((_S, _E), jnp.float32),
    )(x, expert_centroids)
    st = scores.T.reshape(_E, _TR, 128)
    slot2d, te2d, nt2d, cb2d, al2d = pl.pallas_call(
        _route_body,
        out_shape=(
            jax.ShapeDtypeStruct((_TR, 128), jnp.int32),
            jax.ShapeDtypeStruct((1, _NT), jnp.int32),
            jax.ShapeDtypeStruct((1, 1), jnp.int32),
            jax.ShapeDtypeStruct((1, _E + 1), jnp.int32),
            jax.ShapeDtypeStruct((_TR, 128), jnp.float32),
        ),
    )(st)
    return (slot2d.reshape(_S), te2d.reshape(_NT), nt2d.reshape(1),
            cb2d.reshape(_E + 1), al2d.reshape(_S, 1))


def _ueff(u, nt):
    return jnp.minimum(u, nt[0] - 1)


def _ln(te, nt, x_pad, ln_g, ln_b):
    grid_spec = pltpu.PrefetchScalarGridSpec(
        num_scalar_prefetch=2,
        grid=(_NT,),
        in_specs=[
            pl.BlockSpec((_TM, _D), lambda u, te, nt: (_ueff(u, nt), 0)),
            pl.BlockSpec((1, 1, _D),
                         lambda u, te, nt: (te[_ueff(u, nt)], 0, 0)),
            pl.BlockSpec((1, 1, _D),
                         lambda u, te, nt: (te[_ueff(u, nt)], 0, 0)),
        ],
        out_specs=pl.BlockSpec((_TM, _D), lambda u, te, nt: (_ueff(u, nt), 0)),
    )
    return pl.pallas_call(
        _ln_body,
        grid_spec=grid_spec,
        out_shape=jax.ShapeDtypeStruct((_NP, _D), jnp.bfloat16),
    )(te, nt, x_pad, ln_g.reshape(_E, 1, _D), ln_b.reshape(_E, 1, _D))


def _ffn(cb, lnb, w1, b1, w2, b2):
    grid_spec = pltpu.PrefetchScalarGridSpec(
        num_scalar_prefetch=1,
        grid=(_NF, _E),
        in_specs=[
            pl.BlockSpec((1, _FT, _D), lambda f, e, cb: (e, f, 0)),
            pl.BlockSpec((1, 1, 1, _FT), lambda f, e, cb: (e, f, 0, 0)),
            pl.BlockSpec((1, _D, _FT), lambda f, e, cb: (e, 0, f)),
            pl.BlockSpec((1, 1, _D), lambda f, e, cb: (e, 0, 0)),
            pl.BlockSpec((_NP, _D), lambda f, e, cb: (0, 0)),
        ],
        out_specs=pl.BlockSpec((_NP, _D), lambda f, e, cb: (0, 0)),
        scratch_shapes=[pltpu.VMEM((_FT, _D), jnp.bfloat16),
                        pltpu.VMEM((_D, _FT), jnp.bfloat16)],
    )
    return pl.pallas_call(
        _ffn_body,
        grid_spec=grid_spec,
        out_shape=jax.ShapeDtypeStruct((_NP, _D), jnp.float32),
        compiler_params=pltpu.CompilerParams(
            dimension_semantics=("arbitrary", "arbitrary"),
            vmem_limit_bytes=60 * 1024 * 1024),
    )(cb, w1, b1.reshape(_E, _NF, 1, _FT), w2, b2.reshape(_E, 1, _D), lnb)


def _resid(x, g, alpha):
    return pl.pallas_call(
        _resid_body,
        grid=(_TR,),
        in_specs=[pl.BlockSpec((_TM, _D), lambda i: (i, 0)),
                  pl.BlockSpec((_TM, _D), lambda i: (i, 0)),
                  pl.BlockSpec((_TM, 1), lambda i: (i, 0))],
        out_specs=pl.BlockSpec((_TM, _D), lambda i: (i, 0)),
        out_shape=jax.ShapeDtypeStruct((_S, _D), jnp.float32),
    )(x, g, alpha)


def kernel(input_features, expert_centroids, ln_g, ln_b, w1, b1, w2, b2):
    x = input_features.reshape(_S, _D)
    slot, te, nt, cb, alpha = _routing(x, expert_centroids)
    x_pad = _dispatch(x, slot)
    lnb = _ln(te, nt, x_pad, ln_g, ln_b)
    ffn_pad = _ffn(cb, lnb, w1, b1, w2, b2)
    g = _combine(ffn_pad, slot)
    y = _resid(x, g, alpha)
    return y.reshape(input_features.shape)
